# Initial kernel scaffold; baseline (speedup 1.0000x reference)
import jax
import jax.numpy as jnp
from jax.experimental import pallas as pl


def kernel(feat, seg, coords):
    raise NotImplementedError("write your pallas kernel here")


# speedup vs baseline: 9.7372x; 9.7372x over previous
import functools

import jax
import jax.numpy as jnp
from jax import lax
from jax.experimental import pallas as pl
from jax.experimental.pallas import tpu as pltpu
from jax.experimental.pallas import tpu_sc as plsc

NB = 8
HALF2D = NB * NB
C_IN = 3
CH = C_IN + 1
INV_S2 = 1.0 / (2.0 * 0.025 * 0.025)
NPTS = 1048576
NSEG = 4096
NW = 32
SEG_W = NSEG // NW
ROW = HALF2D * CH
CHUNK = 2048
PAD = 64
BUF = CHUNK + PAD
NSTEP = CHUNK // 16
T1_W = SEG_W * ROW


def _sc_extractor(feat, seg, coords):
    mesh = plsc.VectorSubcoreMesh(core_axis_name="c", subcore_axis_name="s")

    @functools.partial(
        pl.kernel,
        mesh=mesh,
        out_type=jax.ShapeDtypeStruct((NSEG * ROW,), jnp.float32),
        compiler_params=pltpu.CompilerParams(needs_layout_passes=False,
                                             use_tc_tiling_on_sc=False),
        scratch_types=[
            pltpu.VMEM((BUF,), jnp.int32),
            pltpu.VMEM((BUF, 2), jnp.float32),
            pltpu.VMEM((BUF, 5), jnp.float32),
            pltpu.VMEM((T1_W,), jnp.float32),
            pltpu.VMEM((SEG_W * HALF2D,), jnp.float32),
            pltpu.VMEM((16 * SEG_W,), jnp.float32),
            pltpu.VMEM((16 * SEG_W,), jnp.float32),
            pltpu.VMEM((16 * SEG_W,), jnp.float32),
            pltpu.VMEM((16 * SEG_W,), jnp.float32),
            pltpu.VMEM((16 * SEG_W,), jnp.float32),
            pltpu.VMEM((SEG_W,), jnp.float32),
            pltpu.VMEM((SEG_W,), jnp.float32),
            pltpu.VMEM((SEG_W,), jnp.float32),
            pltpu.VMEM((SEG_W,), jnp.float32),
            pltpu.VMEM((SEG_W,), jnp.float32),
            pltpu.VMEM((16,), jnp.int32),
        ],
    )
    def k(feat_hbm, seg_hbm, coords_hbm, out_hbm,
          segb, cb, fb, t1, wacc, lox, loy, hix, hiy, cnt16,
          loxr, loyr, dxr, dyr, cntr, pbuf):
        wid = lax.axis_index("s") * 2 + lax.axis_index("c")
        seg_base = wid * SEG_W
        lane = lax.iota(jnp.int32, 16)
        zf = jnp.zeros((16,), jnp.float32)
        onesf = jnp.ones((16,), jnp.float32)
        zi = jnp.zeros((16,), jnp.int32)
        onesi = jnp.ones((16,), jnp.int32)
        big = jnp.full((16,), 3e38, jnp.float32)

        def lower_bound(target):
            def cond(c):
                return c[0] < c[1]

            def body(c):
                lo, hi = c
                mid = lax.div(lo + hi, 2)
                m8 = jnp.minimum(lax.div(mid, 8) * 8, NPTS - 16)
                pltpu.sync_copy(seg_hbm.at[pl.ds(pl.multiple_of(m8, 8), 16)],
                                pbuf)
                v = lax.reduce_max(
                    plsc.load_gather(pbuf, [jnp.full((16,), mid - m8,
                                                     jnp.int32)]),
                    axes=(0,))
                take = v >= target
                return (jnp.where(take, lo, mid + 1),
                        jnp.where(take, mid, hi))

            return lax.while_loop(cond, body, (0, NPTS))[0]

        p0 = lower_bound(seg_base)
        p1 = lower_bound(seg_base + SEG_W)
        n = p1 - p0
        nchunks = lax.div(n + (CHUNK - 1), CHUNK)

        def z_t1(i, _):
            t1[pl.ds(pl.multiple_of(i * 16, 16), 16)] = zf
            return 0
        lax.fori_loop(0, T1_W // 16, z_t1, 0)

        def z_wacc(i, _):
            wacc[pl.ds(pl.multiple_of(i * 16, 16), 16)] = zf
            return 0
        lax.fori_loop(0, SEG_W * HALF2D // 16, z_wacc, 0)

        def z_lp(i, _):
            sl = pl.ds(pl.multiple_of(i * 16, 16), 16)
            lox[sl] = big
            loy[sl] = big
            hix[sl] = -big
            hiy[sl] = -big
            cnt16[sl] = zf
            return 0
        lax.fori_loop(0, 16 * SEG_W // 16, z_lp, 0)

        def chunk1(kk, _):
            p = p0 + kk * CHUNK
            b8 = jnp.minimum(lax.div(p, 8) * 8, NPTS - BUF)
            b8 = pl.multiple_of(b8, 8)
            shift = p - b8
            pltpu.sync_copy(seg_hbm.at[pl.ds(b8, BUF)], segb)
            pltpu.sync_copy(coords_hbm.at[pl.ds(b8, BUF)], cb)
            rem = jnp.minimum(p1 - p, CHUNK)

            def step(j, _):
                rel = lane * NSTEP + j
                valid = rel < rem
                ib = jnp.minimum(rel + shift, BUF - 1)
                sv = plsc.load_gather(segb, [ib])
                sl = jnp.clip(sv - seg_base, 0, SEG_W - 1)
                tidx = lane * SEG_W + sl
                cx = plsc.load_gather(cb, [ib, zi])
                cy = plsc.load_gather(cb, [ib, onesi])
                cur = plsc.load_gather(lox, [tidx])
                plsc.store_scatter(lox, [tidx], jnp.minimum(cur, cx),
                                   mask=valid)
                cur = plsc.load_gather(loy, [tidx])
                plsc.store_scatter(loy, [tidx], jnp.minimum(cur, cy),
                                   mask=valid)
                cur = plsc.load_gather(hix, [tidx])
                plsc.store_scatter(hix, [tidx], jnp.maximum(cur, cx),
                                   mask=valid)
                cur = plsc.load_gather(hiy, [tidx])
                plsc.store_scatter(hiy, [tidx], jnp.maximum(cur, cy),
                                   mask=valid)
                plsc.addupdate_scatter(cnt16, [tidx], onesf, mask=valid)
                return 0

            lax.fori_loop(0, NSTEP, step, 0)
            return 0

        lax.fori_loop(0, nchunks, chunk1, 0)

        for blk in range(SEG_W // 16):
            o = blk * 16
            mnx = lox[pl.ds(o, 16)]
            mny = loy[pl.ds(o, 16)]
            mxx = hix[pl.ds(o, 16)]
            mxy = hiy[pl.ds(o, 16)]
            csum = cnt16[pl.ds(o, 16)]
            for l in range(1, 16):
                q = l * SEG_W + o
                mnx = jnp.minimum(mnx, lox[pl.ds(q, 16)])
                mny = jnp.minimum(mny, loy[pl.ds(q, 16)])
                mxx = jnp.maximum(mxx, hix[pl.ds(q, 16)])
                mxy = jnp.maximum(mxy, hiy[pl.ds(q, 16)])
                csum = csum + cnt16[pl.ds(q, 16)]
            loxr[pl.ds(o, 16)] = mnx
            loyr[pl.ds(o, 16)] = mny
            dxr[pl.ds(o, 16)] = mxx - mnx + 1e-6
            dyr[pl.ds(o, 16)] = mxy - mny + 1e-6
            cntr[pl.ds(o, 16)] = csum

        c3 = jnp.full((16,), 3, jnp.int32)
        c4 = jnp.full((16,), 4, jnp.int32)
        c0 = zi
        c1 = onesi
        c2 = jnp.full((16,), 2, jnp.int32)

        def chunk2(kk, _):
            p = p0 + kk * CHUNK
            b8 = jnp.minimum(lax.div(p, 8) * 8, NPTS - BUF)
            b8 = pl.multiple_of(b8, 8)
            shift = p - b8
            pltpu.sync_copy(seg_hbm.at[pl.ds(b8, BUF)], segb)
            pltpu.sync_copy(coords_hbm.at[pl.ds(b8, BUF)], cb)
            pltpu.sync_copy(feat_hbm.at[pl.ds(b8, BUF)], fb)
            rem = jnp.minimum(p1 - p, CHUNK)

            def step(j, _):
                rel = j * 16 + lane
                valid = rel < rem
                ib = jnp.minimum(rel + shift, BUF - 1)
                sv = plsc.load_gather(segb, [ib])
                sl = jnp.clip(sv - seg_base, 0, SEG_W - 1)
                cx = plsc.load_gather(cb, [ib, zi])
                cy = plsc.load_gather(cb, [ib, onesi])
                lx = plsc.load_gather(loxr, [sl])
                ly = plsc.load_gather(loyr, [sl])
                dx = plsc.load_gather(dxr, [sl])
                dy = plsc.load_gather(dyr, [sl])
                gx = (cx - lx) / dx * 7.0
                gy = (cy - ly) / dy * 7.0
                i0x = jnp.minimum(gx.astype(jnp.int32), 6)
                i0y = jnp.minimum(gy.astype(jnp.int32), 6)
                fx = gx - i0x.astype(jnp.float32)
                fy = gy - i0y.astype(jnp.float32)
                wx0 = 1.0 - fx
                wy0 = 1.0 - fy
                w00 = wx0 * wy0
                w01 = wx0 * fy
                w10 = fx * wy0
                w11 = fx * fy
                base = sl * ROW + i0x * (NB * CH) + i0y * CH
                f0 = plsc.load_gather(fb, [ib, c0])
                f1 = plsc.load_gather(fb, [ib, c1])
                f2 = plsc.load_gather(fb, [ib, c2])
                for off, w in ((0, w00), (CH, w01), (NB * CH, w10),
                               (NB * CH + CH, w11)):
                    bb = base + off
                    plsc.addupdate_scatter(t1, [bb], f0 * w, mask=valid)
                    plsc.addupdate_scatter(t1, [bb + 1], f1 * w, mask=valid)
                    plsc.addupdate_scatter(t1, [bb + 2], f2 * w, mask=valid)
                    plsc.addupdate_scatter(wacc, [lax.shift_right_logical(bb, 2)],
                                           w, mask=valid)
                f3 = plsc.load_gather(fb, [ib, c3])
                f4 = plsc.load_gather(fb, [ib, c4])
                g3 = f3 * 8.0 - 0.5
                g4 = f4 * 8.0 - 0.5
                i3f = (g3 + 8.0).astype(jnp.int32).astype(jnp.float32) - 8.0
                i4f = (g4 + 8.0).astype(jnp.int32).astype(jnp.float32) - 8.0
                fr3 = g3 - i3f
                fr4 = g4 - i4f
                wa3 = 1.0 / (1.0 + jnp.exp((2.0 * fr3 - 1.0) * INV_S2))
                wa4 = 1.0 / (1.0 + jnp.exp((2.0 * fr4 - 1.0) * INV_S2))
                wb3 = 1.0 - wa3
                wb4 = 1.0 - wa4
                i3 = i3f.astype(jnp.int32)
                i4 = i4f.astype(jnp.int32)
                a3 = jnp.clip(i3, 0, NB - 1) * (NB * CH)
                b3 = jnp.clip(i3 + 1, 0, NB - 1) * (NB * CH)
                a4 = jnp.clip(i4, 0, NB - 1) * CH
                b4 = jnp.clip(i4 + 1, 0, NB - 1) * CH
                hb = sl * ROW + 3
                plsc.addupdate_scatter(t1, [hb + a3 + a4], wa3 * wa4,
                                       mask=valid)
                plsc.addupdate_scatter(t1, [hb + a3 + b4], wa3 * wb4,
                                       mask=valid)
                plsc.addupdate_scatter(t1, [hb + b3 + a4], wb3 * wa4,
                                       mask=valid)
                plsc.addupdate_scatter(t1, [hb + b3 + b4], wb3 * wb4,
                                       mask=valid)
                return 0

            lax.fori_loop(0, NSTEP, step, 0)
            return 0

        lax.fori_loop(0, nchunks, chunk2, 0)

        ch_is_hist = (lane & 3) == 3

        def fin(kk, _):
            idxv = kk * 16 + lane
            wv = plsc.load_gather(wacc, [lax.shift_right_logical(idxv, 2)])
            cv = plsc.load_gather(cntr, [lax.shift_right_logical(idxv, 8)])
            den = jnp.where(ch_is_hist, cv, wv) + 1e-6
            v = plsc.load_gather(t1, [idxv])
            plsc.store_scatter(t1, [idxv], v / den)
            return 0

        lax.fori_loop(0, T1_W // 16, fin, 0)
        pltpu.sync_copy(t1, out_hbm.at[pl.ds(wid * T1_W, T1_W)])

    return k(feat, seg, coords)


def kernel(feat, seg, coords):
    out = _sc_extractor(feat, seg.astype(jnp.int32), coords)
    return out.reshape(NSEG, ROW)


# speedup vs baseline: 12.4609x; 1.2797x over previous
import functools

import jax
import jax.numpy as jnp
from jax import lax
from jax.experimental import pallas as pl
from jax.experimental.pallas import tpu as pltpu
from jax.experimental.pallas import tpu_sc as plsc

NB = 8
HALF2D = NB * NB
C_IN = 3
CH = C_IN + 1
INV_S2 = 1.0 / (2.0 * 0.025 * 0.025)
NPTS = 1048576
NSEG = 4096
NW = 32
SEG_W = NSEG // NW
ROW = HALF2D * CH
CHUNK = 2048
PAD = 64
BUF = CHUNK + PAD
NSTEP = CHUNK // 16
T1_W = SEG_W * ROW


def _sc_extractor(feat, seg, coords):
    mesh = plsc.VectorSubcoreMesh(core_axis_name="c", subcore_axis_name="s")

    @functools.partial(
        pl.kernel,
        mesh=mesh,
        out_type=jax.ShapeDtypeStruct((NSEG * ROW,), jnp.float32),
        compiler_params=pltpu.CompilerParams(needs_layout_passes=False,
                                             use_tc_tiling_on_sc=False),
        scratch_types=[
            pltpu.VMEM((BUF,), jnp.int32),
            pltpu.VMEM((BUF * 2,), jnp.float32),
            pltpu.VMEM((BUF * 5,), jnp.float32),
            pltpu.VMEM((T1_W,), jnp.float32),
            pltpu.VMEM((SEG_W * HALF2D,), jnp.float32),
            pltpu.VMEM((16 * SEG_W,), jnp.float32),
            pltpu.VMEM((16 * SEG_W,), jnp.float32),
            pltpu.VMEM((16 * SEG_W,), jnp.float32),
            pltpu.VMEM((16 * SEG_W,), jnp.float32),
            pltpu.VMEM((16 * SEG_W,), jnp.float32),
            pltpu.VMEM((SEG_W,), jnp.float32),
            pltpu.VMEM((SEG_W,), jnp.float32),
            pltpu.VMEM((SEG_W,), jnp.float32),
            pltpu.VMEM((SEG_W,), jnp.float32),
            pltpu.VMEM((SEG_W,), jnp.float32),
            pltpu.VMEM((16,), jnp.int32),
        ],
    )
    def k(feat_hbm, seg_hbm, coords_hbm, out_hbm,
          segb, cb, fb, t1, wacc, lox, loy, hix, hiy, cnt16,
          loxr, loyr, dxr, dyr, cntr, pbuf):
        wid = lax.axis_index("s") * 2 + lax.axis_index("c")
        seg_base = wid * SEG_W
        lane = lax.iota(jnp.int32, 16)
        zf = jnp.zeros((16,), jnp.float32)
        onesf = jnp.ones((16,), jnp.float32)
        zi = jnp.zeros((16,), jnp.int32)
        onesi = jnp.ones((16,), jnp.int32)
        big = jnp.full((16,), 3e38, jnp.float32)

        def lower_bound(target):
            def cond(c):
                return c[0] < c[1]

            def body(c):
                lo, hi = c
                mid = lax.div(lo + hi, 2)
                m8 = jnp.minimum(lax.div(mid, 8) * 8, NPTS - 16)
                pltpu.sync_copy(seg_hbm.at[pl.ds(pl.multiple_of(m8, 8), 16)],
                                pbuf)
                v = lax.reduce_max(
                    plsc.load_gather(pbuf, [jnp.full((16,), mid - m8,
                                                     jnp.int32)]),
                    axes=(0,))
                take = v >= target
                return (jnp.where(take, lo, mid + 1),
                        jnp.where(take, mid, hi))

            return lax.while_loop(cond, body, (0, NPTS))[0]

        p0 = lower_bound(seg_base)
        p1 = lower_bound(seg_base + SEG_W)
        n = p1 - p0
        nchunks = lax.div(n + (CHUNK - 1), CHUNK)

        def z_t1(i, _):
            t1[pl.ds(pl.multiple_of(i * 16, 16), 16)] = zf
            return 0
        lax.fori_loop(0, T1_W // 16, z_t1, 0)

        def z_wacc(i, _):
            wacc[pl.ds(pl.multiple_of(i * 16, 16), 16)] = zf
            return 0
        lax.fori_loop(0, SEG_W * HALF2D // 16, z_wacc, 0)

        def z_lp(i, _):
            sl = pl.ds(pl.multiple_of(i * 16, 16), 16)
            lox[sl] = big
            loy[sl] = big
            hix[sl] = -big
            hiy[sl] = -big
            cnt16[sl] = zf
            return 0
        lax.fori_loop(0, 16 * SEG_W // 16, z_lp, 0)

        def chunk1(kk, _):
            p = p0 + kk * CHUNK
            b8 = jnp.minimum(lax.div(p, 8) * 8, NPTS - BUF)
            b8 = pl.multiple_of(b8, 8)
            shift = p - b8
            pltpu.sync_copy(seg_hbm.at[pl.ds(b8, BUF)], segb)
            pltpu.sync_copy(
                coords_hbm.at[pl.ds(pl.multiple_of(b8 * 2, 16), BUF * 2)], cb)
            rem = jnp.minimum(p1 - p, CHUNK)

            def step(j, _):
                rel = lane * NSTEP + j
                valid = rel < rem
                ib = jnp.minimum(rel + shift, BUF - 1)
                sv = plsc.load_gather(segb, [ib])
                sl = jnp.clip(sv - seg_base, 0, SEG_W - 1)
                tidx = lane * SEG_W + sl
                ib2 = ib * 2
                cx = plsc.load_gather(cb, [ib2])
                cy = plsc.load_gather(cb, [ib2 + 1])
                cur = plsc.load_gather(lox, [tidx])
                plsc.store_scatter(lox, [tidx], jnp.minimum(cur, cx),
                                   mask=valid)
                cur = plsc.load_gather(loy, [tidx])
                plsc.store_scatter(loy, [tidx], jnp.minimum(cur, cy),
                                   mask=valid)
                cur = plsc.load_gather(hix, [tidx])
                plsc.store_scatter(hix, [tidx], jnp.maximum(cur, cx),
                                   mask=valid)
                cur = plsc.load_gather(hiy, [tidx])
                plsc.store_scatter(hiy, [tidx], jnp.maximum(cur, cy),
                                   mask=valid)
                plsc.addupdate_scatter(cnt16, [tidx], onesf, mask=valid)
                return 0

            lax.fori_loop(0, NSTEP, step, 0)
            return 0

        lax.fori_loop(0, nchunks, chunk1, 0)

        for blk in range(SEG_W // 16):
            o = blk * 16
            mnx = lox[pl.ds(o, 16)]
            mny = loy[pl.ds(o, 16)]
            mxx = hix[pl.ds(o, 16)]
            mxy = hiy[pl.ds(o, 16)]
            csum = cnt16[pl.ds(o, 16)]
            for l in range(1, 16):
                q = l * SEG_W + o
                mnx = jnp.minimum(mnx, lox[pl.ds(q, 16)])
                mny = jnp.minimum(mny, loy[pl.ds(q, 16)])
                mxx = jnp.maximum(mxx, hix[pl.ds(q, 16)])
                mxy = jnp.maximum(mxy, hiy[pl.ds(q, 16)])
                csum = csum + cnt16[pl.ds(q, 16)]
            loxr[pl.ds(o, 16)] = mnx
            loyr[pl.ds(o, 16)] = mny
            dxr[pl.ds(o, 16)] = mxx - mnx + 1e-6
            dyr[pl.ds(o, 16)] = mxy - mny + 1e-6
            cntr[pl.ds(o, 16)] = csum

        def chunk2(kk, _):
            p = p0 + kk * CHUNK
            b8 = jnp.minimum(lax.div(p, 8) * 8, NPTS - BUF)
            b8 = pl.multiple_of(b8, 8)
            shift = p - b8
            pltpu.sync_copy(seg_hbm.at[pl.ds(b8, BUF)], segb)
            pltpu.sync_copy(
                coords_hbm.at[pl.ds(pl.multiple_of(b8 * 2, 16), BUF * 2)], cb)
            pltpu.sync_copy(
                feat_hbm.at[pl.ds(pl.multiple_of(b8 * 5, 40), BUF * 5)], fb)
            rem = jnp.minimum(p1 - p, CHUNK)

            def step(j, _):
                rel = j * 16 + lane
                valid = rel < rem
                ib = jnp.minimum(rel + shift, BUF - 1)
                sv = plsc.load_gather(segb, [ib])
                sl = jnp.clip(sv - seg_base, 0, SEG_W - 1)
                ib2 = ib * 2
                cx = plsc.load_gather(cb, [ib2])
                cy = plsc.load_gather(cb, [ib2 + 1])
                lx = plsc.load_gather(loxr, [sl])
                ly = plsc.load_gather(loyr, [sl])
                dx = plsc.load_gather(dxr, [sl])
                dy = plsc.load_gather(dyr, [sl])
                gx = (cx - lx) / dx * 7.0
                gy = (cy - ly) / dy * 7.0
                i0x = jnp.minimum(gx.astype(jnp.int32), 6)
                i0y = jnp.minimum(gy.astype(jnp.int32), 6)
                fx = gx - i0x.astype(jnp.float32)
                fy = gy - i0y.astype(jnp.float32)
                wx0 = 1.0 - fx
                wy0 = 1.0 - fy
                w00 = wx0 * wy0
                w01 = wx0 * fy
                w10 = fx * wy0
                w11 = fx * fy
                base = sl * ROW + i0x * (NB * CH) + i0y * CH
                ib5 = ib * 5
                f0 = plsc.load_gather(fb, [ib5])
                f1 = plsc.load_gather(fb, [ib5 + 1])
                f2 = plsc.load_gather(fb, [ib5 + 2])
                for off, w in ((0, w00), (CH, w01), (NB * CH, w10),
                               (NB * CH + CH, w11)):
                    bb = base + off
                    plsc.addupdate_scatter(t1, [bb], f0 * w, mask=valid)
                    plsc.addupdate_scatter(t1, [bb + 1], f1 * w, mask=valid)
                    plsc.addupdate_scatter(t1, [bb + 2], f2 * w, mask=valid)
                    plsc.addupdate_scatter(wacc, [lax.shift_right_logical(bb, 2)],
                                           w, mask=valid)
                f3 = plsc.load_gather(fb, [ib5 + 3])
                f4 = plsc.load_gather(fb, [ib5 + 4])
                g3 = f3 * 8.0 - 0.5
                g4 = f4 * 8.0 - 0.5
                i3f = (g3 + 8.0).astype(jnp.int32).astype(jnp.float32) - 8.0
                i4f = (g4 + 8.0).astype(jnp.int32).astype(jnp.float32) - 8.0
                fr3 = g3 - i3f
                fr4 = g4 - i4f
                wa3 = 1.0 / (1.0 + jnp.exp((2.0 * fr3 - 1.0) * INV_S2))
                wa4 = 1.0 / (1.0 + jnp.exp((2.0 * fr4 - 1.0) * INV_S2))
                wb3 = 1.0 - wa3
                wb4 = 1.0 - wa4
                i3 = i3f.astype(jnp.int32)
                i4 = i4f.astype(jnp.int32)
                a3 = jnp.clip(i3, 0, NB - 1) * (NB * CH)
                b3 = jnp.clip(i3 + 1, 0, NB - 1) * (NB * CH)
                a4 = jnp.clip(i4, 0, NB - 1) * CH
                b4 = jnp.clip(i4 + 1, 0, NB - 1) * CH
                hb = sl * ROW + 3
                plsc.addupdate_scatter(t1, [hb + a3 + a4], wa3 * wa4,
                                       mask=valid)
                plsc.addupdate_scatter(t1, [hb + a3 + b4], wa3 * wb4,
                                       mask=valid)
                plsc.addupdate_scatter(t1, [hb + b3 + a4], wb3 * wa4,
                                       mask=valid)
                plsc.addupdate_scatter(t1, [hb + b3 + b4], wb3 * wb4,
                                       mask=valid)
                return 0

            lax.fori_loop(0, NSTEP, step, 0)
            return 0

        lax.fori_loop(0, nchunks, chunk2, 0)

        ch_is_hist = (lane & 3) == 3

        def fin(kk, _):
            idxv = kk * 16 + lane
            wv = plsc.load_gather(wacc, [lax.shift_right_logical(idxv, 2)])
            cv = plsc.load_gather(cntr, [lax.shift_right_logical(idxv, 8)])
            den = jnp.where(ch_is_hist, cv, wv) + 1e-6
            v = plsc.load_gather(t1, [idxv])
            plsc.store_scatter(t1, [idxv], v / den)
            return 0

        lax.fori_loop(0, T1_W // 16, fin, 0)
        pltpu.sync_copy(t1, out_hbm.at[pl.ds(wid * T1_W, T1_W)])

    return k(feat, seg, coords)


def kernel(feat, seg, coords):
    out = _sc_extractor(feat.reshape(-1), seg.astype(jnp.int32),
                        coords.reshape(-1))
    return out.reshape(NSEG, ROW)


# speedup vs baseline: 45.3881x; 3.6425x over previous
import functools

import jax
import jax.numpy as jnp
from jax import lax
from jax.experimental import pallas as pl
from jax.experimental.pallas import tpu as pltpu
from jax.experimental.pallas import tpu_sc as plsc

NB = 8
HALF2D = NB * NB
C_IN = 3
CH = C_IN + 1
INV_S2 = 1.0 / (2.0 * 0.025 * 0.025)
NPTS = 1048576
NSEG = 4096
NW = 32
SEG_W = NSEG // NW
ROW = HALF2D * CH
CHUNK = 2048
PAD = 64
BUF = CHUNK + PAD
NSTEP = CHUNK // 16
T1_W = SEG_W * ROW


def _sc_extractor(f0a, f1a, f2a, f3a, f4a, cxa, cya, seg):
    mesh = plsc.VectorSubcoreMesh(core_axis_name="c", subcore_axis_name="s")

    @functools.partial(
        pl.kernel,
        mesh=mesh,
        out_type=jax.ShapeDtypeStruct((NSEG * ROW,), jnp.float32),
        compiler_params=pltpu.CompilerParams(needs_layout_passes=False,
                                             use_tc_tiling_on_sc=False),
        scratch_types=[
            pltpu.VMEM((BUF,), jnp.int32),
            pltpu.VMEM((BUF,), jnp.float32),
            pltpu.VMEM((BUF,), jnp.float32),
            pltpu.VMEM((BUF,), jnp.float32),
            pltpu.VMEM((BUF,), jnp.float32),
            pltpu.VMEM((BUF,), jnp.float32),
            pltpu.VMEM((BUF,), jnp.float32),
            pltpu.VMEM((BUF,), jnp.float32),
            pltpu.VMEM((T1_W,), jnp.float32),
            pltpu.VMEM((SEG_W * HALF2D,), jnp.float32),
            pltpu.VMEM((16 * SEG_W,), jnp.float32),
            pltpu.VMEM((16 * SEG_W,), jnp.float32),
            pltpu.VMEM((16 * SEG_W,), jnp.float32),
            pltpu.VMEM((16 * SEG_W,), jnp.float32),
            pltpu.VMEM((16 * SEG_W,), jnp.float32),
            pltpu.VMEM((SEG_W,), jnp.float32),
            pltpu.VMEM((SEG_W,), jnp.float32),
            pltpu.VMEM((SEG_W,), jnp.float32),
            pltpu.VMEM((SEG_W,), jnp.float32),
            pltpu.VMEM((SEG_W,), jnp.float32),
            pltpu.VMEM((16,), jnp.int32),
        ],
    )
    def k(f0h, f1h, f2h, f3h, f4h, cxh, cyh, seg_hbm, out_hbm,
          segb, cxb, cyb, f0b, f1b, f2b, f3b, f4b,
          t1, wacc, lox, loy, hix, hiy, cnt16,
          loxr, loyr, dxr, dyr, cntr, pbuf):
        wid = lax.axis_index("s") * 2 + lax.axis_index("c")
        seg_base = wid * SEG_W
        lane = lax.iota(jnp.int32, 16)
        zf = jnp.zeros((16,), jnp.float32)
        onesf = jnp.ones((16,), jnp.float32)
        zi = jnp.zeros((16,), jnp.int32)
        onesi = jnp.ones((16,), jnp.int32)
        big = jnp.full((16,), 3e38, jnp.float32)

        def lower_bound(target):
            def cond(c):
                return c[0] < c[1]

            def body(c):
                lo, hi = c
                mid = lax.div(lo + hi, 2)
                m8 = jnp.minimum(lax.div(mid, 8) * 8, NPTS - 16)
                pltpu.sync_copy(seg_hbm.at[pl.ds(pl.multiple_of(m8, 8), 16)],
                                pbuf)
                v = lax.reduce_max(
                    plsc.load_gather(pbuf, [jnp.full((16,), mid - m8,
                                                     jnp.int32)]),
                    axes=(0,))
                take = v >= target
                return (jnp.where(take, lo, mid + 1),
                        jnp.where(take, mid, hi))

            return lax.while_loop(cond, body, (0, NPTS))[0]

        p0 = lower_bound(seg_base)
        p1 = lower_bound(seg_base + SEG_W)
        n = p1 - p0
        nchunks = lax.div(n + (CHUNK - 1), CHUNK)

        def z_t1(i, _):
            t1[pl.ds(pl.multiple_of(i * 16, 16), 16)] = zf
            return 0
        lax.fori_loop(0, T1_W // 16, z_t1, 0)

        def z_wacc(i, _):
            wacc[pl.ds(pl.multiple_of(i * 16, 16), 16)] = zf
            return 0
        lax.fori_loop(0, SEG_W * HALF2D // 16, z_wacc, 0)

        def z_lp(i, _):
            sl = pl.ds(pl.multiple_of(i * 16, 16), 16)
            lox[sl] = big
            loy[sl] = big
            hix[sl] = -big
            hiy[sl] = -big
            cnt16[sl] = zf
            return 0
        lax.fori_loop(0, 16 * SEG_W // 16, z_lp, 0)

        def chunk1(kk, _):
            p = p0 + kk * CHUNK
            b8 = jnp.minimum(lax.div(p, 8) * 8, NPTS - BUF)
            b8 = pl.multiple_of(b8, 8)
            shift = p - b8
            pltpu.sync_copy(seg_hbm.at[pl.ds(b8, BUF)], segb)
            pltpu.sync_copy(cxh.at[pl.ds(b8, BUF)], cxb)
            pltpu.sync_copy(cyh.at[pl.ds(b8, BUF)], cyb)
            rem = jnp.minimum(p1 - p, CHUNK)

            def step(j, _):
                rel = lane * NSTEP + j
                valid = rel < rem
                ib = jnp.minimum(rel + shift, BUF - 1)
                sv = plsc.load_gather(segb, [ib])
                sl = jnp.clip(sv - seg_base, 0, SEG_W - 1)
                tidx = lane * SEG_W + sl
                cx = plsc.load_gather(cxb, [ib])
                cy = plsc.load_gather(cyb, [ib])
                cur = plsc.load_gather(lox, [tidx])
                plsc.store_scatter(lox, [tidx], jnp.minimum(cur, cx),
                                   mask=valid)
                cur = plsc.load_gather(loy, [tidx])
                plsc.store_scatter(loy, [tidx], jnp.minimum(cur, cy),
                                   mask=valid)
                cur = plsc.load_gather(hix, [tidx])
                plsc.store_scatter(hix, [tidx], jnp.maximum(cur, cx),
                                   mask=valid)
                cur = plsc.load_gather(hiy, [tidx])
                plsc.store_scatter(hiy, [tidx], jnp.maximum(cur, cy),
                                   mask=valid)
                plsc.addupdate_scatter(cnt16, [tidx], onesf, mask=valid)
                return 0

            lax.fori_loop(0, NSTEP, step, 0)
            return 0

        lax.fori_loop(0, nchunks, chunk1, 0)

        for blk in range(SEG_W // 16):
            o = blk * 16
            mnx = lox[pl.ds(o, 16)]
            mny = loy[pl.ds(o, 16)]
            mxx = hix[pl.ds(o, 16)]
            mxy = hiy[pl.ds(o, 16)]
            csum = cnt16[pl.ds(o, 16)]
            for l in range(1, 16):
                q = l * SEG_W + o
                mnx = jnp.minimum(mnx, lox[pl.ds(q, 16)])
                mny = jnp.minimum(mny, loy[pl.ds(q, 16)])
                mxx = jnp.maximum(mxx, hix[pl.ds(q, 16)])
                mxy = jnp.maximum(mxy, hiy[pl.ds(q, 16)])
                csum = csum + cnt16[pl.ds(q, 16)]
            loxr[pl.ds(o, 16)] = mnx
            loyr[pl.ds(o, 16)] = mny
            dxr[pl.ds(o, 16)] = mxx - mnx + 1e-6
            dyr[pl.ds(o, 16)] = mxy - mny + 1e-6
            cntr[pl.ds(o, 16)] = csum

        def chunk2(kk, _):
            p = p0 + kk * CHUNK
            b8 = jnp.minimum(lax.div(p, 8) * 8, NPTS - BUF)
            b8 = pl.multiple_of(b8, 8)
            shift = p - b8
            pltpu.sync_copy(seg_hbm.at[pl.ds(b8, BUF)], segb)
            pltpu.sync_copy(cxh.at[pl.ds(b8, BUF)], cxb)
            pltpu.sync_copy(cyh.at[pl.ds(b8, BUF)], cyb)
            pltpu.sync_copy(f0h.at[pl.ds(b8, BUF)], f0b)
            pltpu.sync_copy(f1h.at[pl.ds(b8, BUF)], f1b)
            pltpu.sync_copy(f2h.at[pl.ds(b8, BUF)], f2b)
            pltpu.sync_copy(f3h.at[pl.ds(b8, BUF)], f3b)
            pltpu.sync_copy(f4h.at[pl.ds(b8, BUF)], f4b)
            rem = jnp.minimum(p1 - p, CHUNK)

            def step(j, _):
                rel = j * 16 + lane
                valid = rel < rem
                ib = jnp.minimum(rel + shift, BUF - 1)
                sv = plsc.load_gather(segb, [ib])
                sl = jnp.clip(sv - seg_base, 0, SEG_W - 1)
                cx = plsc.load_gather(cxb, [ib])
                cy = plsc.load_gather(cyb, [ib])
                lx = plsc.load_gather(loxr, [sl])
                ly = plsc.load_gather(loyr, [sl])
                dx = plsc.load_gather(dxr, [sl])
                dy = plsc.load_gather(dyr, [sl])
                gx = (cx - lx) / dx * 7.0
                gy = (cy - ly) / dy * 7.0
                i0x = jnp.minimum(gx.astype(jnp.int32), 6)
                i0y = jnp.minimum(gy.astype(jnp.int32), 6)
                fx = gx - i0x.astype(jnp.float32)
                fy = gy - i0y.astype(jnp.float32)
                wx0 = 1.0 - fx
                wy0 = 1.0 - fy
                w00 = wx0 * wy0
                w01 = wx0 * fy
                w10 = fx * wy0
                w11 = fx * fy
                base = sl * ROW + i0x * (NB * CH) + i0y * CH
                f0 = plsc.load_gather(f0b, [ib])
                f1 = plsc.load_gather(f1b, [ib])
                f2 = plsc.load_gather(f2b, [ib])
                for off, w in ((0, w00), (CH, w01), (NB * CH, w10),
                               (NB * CH + CH, w11)):
                    bb = base + off
                    plsc.addupdate_scatter(t1, [bb], f0 * w, mask=valid)
                    plsc.addupdate_scatter(t1, [bb + 1], f1 * w, mask=valid)
                    plsc.addupdate_scatter(t1, [bb + 2], f2 * w, mask=valid)
                    plsc.addupdate_scatter(wacc, [lax.shift_right_logical(bb, 2)],
                                           w, mask=valid)
                f3 = plsc.load_gather(f3b, [ib])
                f4 = plsc.load_gather(f4b, [ib])
                g3 = f3 * 8.0 - 0.5
                g4 = f4 * 8.0 - 0.5
                i3f = (g3 + 8.0).astype(jnp.int32).astype(jnp.float32) - 8.0
                i4f = (g4 + 8.0).astype(jnp.int32).astype(jnp.float32) - 8.0
                fr3 = g3 - i3f
                fr4 = g4 - i4f
                wa3 = 1.0 / (1.0 + jnp.exp((2.0 * fr3 - 1.0) * INV_S2))
                wa4 = 1.0 / (1.0 + jnp.exp((2.0 * fr4 - 1.0) * INV_S2))
                wb3 = 1.0 - wa3
                wb4 = 1.0 - wa4
                i3 = i3f.astype(jnp.int32)
                i4 = i4f.astype(jnp.int32)
                a3 = jnp.clip(i3, 0, NB - 1) * (NB * CH)
                b3 = jnp.clip(i3 + 1, 0, NB - 1) * (NB * CH)
                a4 = jnp.clip(i4, 0, NB - 1) * CH
                b4 = jnp.clip(i4 + 1, 0, NB - 1) * CH
                hb = sl * ROW + 3
                plsc.addupdate_scatter(t1, [hb + a3 + a4], wa3 * wa4,
                                       mask=valid)
                plsc.addupdate_scatter(t1, [hb + a3 + b4], wa3 * wb4,
                                       mask=valid)
                plsc.addupdate_scatter(t1, [hb + b3 + a4], wb3 * wa4,
                                       mask=valid)
                plsc.addupdate_scatter(t1, [hb + b3 + b4], wb3 * wb4,
                                       mask=valid)
                return 0

            lax.fori_loop(0, NSTEP, step, 0)
            return 0

        lax.fori_loop(0, nchunks, chunk2, 0)

        ch_is_hist = (lane & 3) == 3

        def fin(kk, _):
            idxv = kk * 16 + lane
            wv = plsc.load_gather(wacc, [lax.shift_right_logical(idxv, 2)])
            cv = plsc.load_gather(cntr, [lax.shift_right_logical(idxv, 8)])
            den = jnp.where(ch_is_hist, cv, wv) + 1e-6
            v = plsc.load_gather(t1, [idxv])
            plsc.store_scatter(t1, [idxv], v / den)
            return 0

        lax.fori_loop(0, T1_W // 16, fin, 0)
        pltpu.sync_copy(t1, out_hbm.at[pl.ds(wid * T1_W, T1_W)])

    return k(f0a, f1a, f2a, f3a, f4a, cxa, cya, seg)


def kernel(feat, seg, coords):
    out = _sc_extractor(feat[:, 0], feat[:, 1], feat[:, 2], feat[:, 3],
                        feat[:, 4], coords[:, 0], coords[:, 1],
                        seg.astype(jnp.int32))
    return out.reshape(NSEG, ROW)


# speedup vs baseline: 49.5444x; 1.0916x over previous
import functools

import jax
import jax.numpy as jnp
from jax import lax
from jax.experimental import pallas as pl
from jax.experimental.pallas import tpu as pltpu
from jax.experimental.pallas import tpu_sc as plsc

NB = 8
HALF2D = NB * NB
C_IN = 3
CH = C_IN + 1
INV_S2 = 1.0 / (2.0 * 0.025 * 0.025)
NPTS = 1048576
NSEG = 4096
NW = 32
SEG_W = NSEG // NW
ROW = HALF2D * CH
CHUNK = 4096
PAD = 64
BUF = CHUNK + PAD
NSTEP = CHUNK // 16
T1_W = SEG_W * ROW


def _sc_extractor(f0a, f1a, f2a, f3a, f4a, cxa, cya, seg):
    mesh = plsc.VectorSubcoreMesh(core_axis_name="c", subcore_axis_name="s")

    @functools.partial(
        pl.kernel,
        mesh=mesh,
        out_type=jax.ShapeDtypeStruct((NSEG * ROW,), jnp.float32),
        compiler_params=pltpu.CompilerParams(needs_layout_passes=False,
                                             use_tc_tiling_on_sc=False),
        scratch_types=[
            pltpu.VMEM((BUF,), jnp.int32),
            pltpu.VMEM((BUF,), jnp.float32),
            pltpu.VMEM((BUF,), jnp.float32),
            pltpu.VMEM((BUF,), jnp.float32),
            pltpu.VMEM((BUF,), jnp.float32),
            pltpu.VMEM((BUF,), jnp.float32),
            pltpu.VMEM((BUF,), jnp.float32),
            pltpu.VMEM((BUF,), jnp.float32),
            pltpu.VMEM((T1_W,), jnp.float32),
            pltpu.VMEM((SEG_W * HALF2D,), jnp.float32),
            pltpu.VMEM((16 * SEG_W,), jnp.float32),
            pltpu.VMEM((16 * SEG_W,), jnp.float32),
            pltpu.VMEM((16 * SEG_W,), jnp.float32),
            pltpu.VMEM((16 * SEG_W,), jnp.float32),
            pltpu.VMEM((16 * SEG_W,), jnp.float32),
            pltpu.VMEM((SEG_W,), jnp.float32),
            pltpu.VMEM((SEG_W,), jnp.float32),
            pltpu.VMEM((SEG_W,), jnp.float32),
            pltpu.VMEM((SEG_W,), jnp.float32),
            pltpu.VMEM((SEG_W,), jnp.float32),
            pltpu.VMEM((1024,), jnp.int32),
            pltpu.VMEM((1024,), jnp.int32),
            pltpu.VMEM((1024,), jnp.int32),
            pltpu.SemaphoreType.DMA,
        ],
    )
    def k(f0h, f1h, f2h, f3h, f4h, cxh, cyh, seg_hbm, out_hbm,
          segb, cxb, cyb, f0b, f1b, f2b, f3b, f4b,
          t1, wacc, lox, loy, hix, hiy, cnt16,
          loxr, loyr, dxr, dyr, cntr, sidx, samp, wind, dmasem):
        wid = lax.axis_index("s") * 2 + lax.axis_index("c")
        seg_base = wid * SEG_W
        lane = lax.iota(jnp.int32, 16)
        zf = jnp.zeros((16,), jnp.float32)
        onesf = jnp.ones((16,), jnp.float32)
        zi = jnp.zeros((16,), jnp.int32)
        onesi = jnp.ones((16,), jnp.int32)
        big = jnp.full((16,), 3e38, jnp.float32)

        def bidx(k, _):
            sidx[pl.ds(pl.multiple_of(k * 16, 16), 16)] = (k * 16 + lane) * 1024
            return 0
        lax.fori_loop(0, 64, bidx, 0)
        pltpu.async_copy(seg_hbm.at[sidx], samp, dmasem).wait()

        def count_lt(buf, t):
            def cbody(k, acc):
                v = buf[pl.ds(pl.multiple_of(k * 16, 16), 16)]
                return acc + jnp.where(v < t, 1, 0).astype(jnp.int32)
            acc = lax.fori_loop(0, 64, cbody, jnp.zeros((16,), jnp.int32),
                                unroll=4)
            return jnp.sum(acc)

        def lower_bound(target):
            c = count_lt(samp, target)
            w = jnp.maximum(c - 1, 0) * 1024
            pltpu.sync_copy(seg_hbm.at[pl.ds(pl.multiple_of(w, 8), 1024)],
                            wind)
            return w + count_lt(wind, target)

        p0 = lower_bound(seg_base)
        p1 = lower_bound(seg_base + SEG_W)
        n = p1 - p0
        nchunks = lax.div(n + (CHUNK - 1), CHUNK)

        def z_t1(i, _):
            t1[pl.ds(pl.multiple_of(i * 16, 16), 16)] = zf
            return 0
        lax.fori_loop(0, T1_W // 16, z_t1, 0, unroll=8)

        def z_wacc(i, _):
            wacc[pl.ds(pl.multiple_of(i * 16, 16), 16)] = zf
            return 0
        lax.fori_loop(0, SEG_W * HALF2D // 16, z_wacc, 0, unroll=8)

        def z_lp(i, _):
            sl = pl.ds(pl.multiple_of(i * 16, 16), 16)
            lox[sl] = big
            loy[sl] = big
            hix[sl] = -big
            hiy[sl] = -big
            cnt16[sl] = zf
            return 0
        lax.fori_loop(0, 16 * SEG_W // 16, z_lp, 0, unroll=4)

        def chunk1(kk, _):
            p = p0 + kk * CHUNK
            b8 = jnp.minimum(lax.div(p, 8) * 8, NPTS - BUF)
            b8 = pl.multiple_of(b8, 8)
            shift = p - b8
            pltpu.sync_copy(seg_hbm.at[pl.ds(b8, BUF)], segb)
            pltpu.sync_copy(cxh.at[pl.ds(b8, BUF)], cxb)
            pltpu.sync_copy(cyh.at[pl.ds(b8, BUF)], cyb)
            rem = jnp.minimum(p1 - p, CHUNK)

            def step(j, _):
                rel = lane * NSTEP + j
                valid = rel < rem
                ib = jnp.minimum(rel + shift, BUF - 1)
                sv = plsc.load_gather(segb, [ib])
                sl = jnp.clip(sv - seg_base, 0, SEG_W - 1)
                tidx = lane * SEG_W + sl
                cx = plsc.load_gather(cxb, [ib])
                cy = plsc.load_gather(cyb, [ib])
                cur = plsc.load_gather(lox, [tidx])
                plsc.store_scatter(lox, [tidx], jnp.minimum(cur, cx),
                                   mask=valid)
                cur = plsc.load_gather(loy, [tidx])
                plsc.store_scatter(loy, [tidx], jnp.minimum(cur, cy),
                                   mask=valid)
                cur = plsc.load_gather(hix, [tidx])
                plsc.store_scatter(hix, [tidx], jnp.maximum(cur, cx),
                                   mask=valid)
                cur = plsc.load_gather(hiy, [tidx])
                plsc.store_scatter(hiy, [tidx], jnp.maximum(cur, cy),
                                   mask=valid)
                plsc.addupdate_scatter(cnt16, [tidx], onesf, mask=valid)
                return 0

            lax.fori_loop(0, NSTEP, step, 0)
            return 0

        lax.fori_loop(0, nchunks, chunk1, 0)

        for blk in range(SEG_W // 16):
            o = blk * 16
            mnx = lox[pl.ds(o, 16)]
            mny = loy[pl.ds(o, 16)]
            mxx = hix[pl.ds(o, 16)]
            mxy = hiy[pl.ds(o, 16)]
            csum = cnt16[pl.ds(o, 16)]
            for l in range(1, 16):
                q = l * SEG_W + o
                mnx = jnp.minimum(mnx, lox[pl.ds(q, 16)])
                mny = jnp.minimum(mny, loy[pl.ds(q, 16)])
                mxx = jnp.maximum(mxx, hix[pl.ds(q, 16)])
                mxy = jnp.maximum(mxy, hiy[pl.ds(q, 16)])
                csum = csum + cnt16[pl.ds(q, 16)]
            loxr[pl.ds(o, 16)] = mnx
            loyr[pl.ds(o, 16)] = mny
            dxr[pl.ds(o, 16)] = mxx - mnx + 1e-6
            dyr[pl.ds(o, 16)] = mxy - mny + 1e-6
            cntr[pl.ds(o, 16)] = csum

        def chunk2(kk, _):
            p = p0 + kk * CHUNK
            b8 = jnp.minimum(lax.div(p, 8) * 8, NPTS - BUF)
            b8 = pl.multiple_of(b8, 8)
            shift = p - b8
            pltpu.sync_copy(seg_hbm.at[pl.ds(b8, BUF)], segb)
            pltpu.sync_copy(cxh.at[pl.ds(b8, BUF)], cxb)
            pltpu.sync_copy(cyh.at[pl.ds(b8, BUF)], cyb)
            pltpu.sync_copy(f0h.at[pl.ds(b8, BUF)], f0b)
            pltpu.sync_copy(f1h.at[pl.ds(b8, BUF)], f1b)
            pltpu.sync_copy(f2h.at[pl.ds(b8, BUF)], f2b)
            pltpu.sync_copy(f3h.at[pl.ds(b8, BUF)], f3b)
            pltpu.sync_copy(f4h.at[pl.ds(b8, BUF)], f4b)
            rem = jnp.minimum(p1 - p, CHUNK)

            def step(j, _):
                rel = j * 16 + lane
                valid = rel < rem
                ib = jnp.minimum(rel + shift, BUF - 1)
                sv = plsc.load_gather(segb, [ib])
                sl = jnp.clip(sv - seg_base, 0, SEG_W - 1)
                cx = plsc.load_gather(cxb, [ib])
                cy = plsc.load_gather(cyb, [ib])
                lx = plsc.load_gather(loxr, [sl])
                ly = plsc.load_gather(loyr, [sl])
                dx = plsc.load_gather(dxr, [sl])
                dy = plsc.load_gather(dyr, [sl])
                gx = (cx - lx) / dx * 7.0
                gy = (cy - ly) / dy * 7.0
                i0x = jnp.minimum(gx.astype(jnp.int32), 6)
                i0y = jnp.minimum(gy.astype(jnp.int32), 6)
                fx = gx - i0x.astype(jnp.float32)
                fy = gy - i0y.astype(jnp.float32)
                wx0 = 1.0 - fx
                wy0 = 1.0 - fy
                w00 = wx0 * wy0
                w01 = wx0 * fy
                w10 = fx * wy0
                w11 = fx * fy
                base = sl * ROW + i0x * (NB * CH) + i0y * CH
                f0 = plsc.load_gather(f0b, [ib])
                f1 = plsc.load_gather(f1b, [ib])
                f2 = plsc.load_gather(f2b, [ib])
                for off, w in ((0, w00), (CH, w01), (NB * CH, w10),
                               (NB * CH + CH, w11)):
                    bb = base + off
                    plsc.addupdate_scatter(t1, [bb], f0 * w, mask=valid)
                    plsc.addupdate_scatter(t1, [bb + 1], f1 * w, mask=valid)
                    plsc.addupdate_scatter(t1, [bb + 2], f2 * w, mask=valid)
                    plsc.addupdate_scatter(wacc, [lax.shift_right_logical(bb, 2)],
                                           w, mask=valid)
                f3 = plsc.load_gather(f3b, [ib])
                f4 = plsc.load_gather(f4b, [ib])
                g3 = f3 * 8.0 - 0.5
                g4 = f4 * 8.0 - 0.5
                i3f = (g3 + 8.0).astype(jnp.int32).astype(jnp.float32) - 8.0
                i4f = (g4 + 8.0).astype(jnp.int32).astype(jnp.float32) - 8.0
                fr3 = g3 - i3f
                fr4 = g4 - i4f
                wa3 = 1.0 / (1.0 + jnp.exp((2.0 * fr3 - 1.0) * INV_S2))
                wa4 = 1.0 / (1.0 + jnp.exp((2.0 * fr4 - 1.0) * INV_S2))
                wb3 = 1.0 - wa3
                wb4 = 1.0 - wa4
                i3 = i3f.astype(jnp.int32)
                i4 = i4f.astype(jnp.int32)
                a3 = jnp.clip(i3, 0, NB - 1) * (NB * CH)
                b3 = jnp.clip(i3 + 1, 0, NB - 1) * (NB * CH)
                a4 = jnp.clip(i4, 0, NB - 1) * CH
                b4 = jnp.clip(i4 + 1, 0, NB - 1) * CH
                hb = sl * ROW + 3
                plsc.addupdate_scatter(t1, [hb + a3 + a4], wa3 * wa4,
                                       mask=valid)
                plsc.addupdate_scatter(t1, [hb + a3 + b4], wa3 * wb4,
                                       mask=valid)
                plsc.addupdate_scatter(t1, [hb + b3 + a4], wb3 * wa4,
                                       mask=valid)
                plsc.addupdate_scatter(t1, [hb + b3 + b4], wb3 * wb4,
                                       mask=valid)
                return 0

            lax.fori_loop(0, NSTEP, step, 0)
            return 0

        lax.fori_loop(0, nchunks, chunk2, 0)

        ch_is_hist = (lane & 3) == 3

        def fin(kk, _):
            idxv = kk * 16 + lane
            wv = plsc.load_gather(wacc, [lax.shift_right_logical(idxv, 2)])
            cv = plsc.load_gather(cntr, [lax.shift_right_logical(idxv, 8)])
            den = jnp.where(ch_is_hist, cv, wv) + 1e-6
            v = plsc.load_gather(t1, [idxv])
            plsc.store_scatter(t1, [idxv], v / den)
            return 0

        lax.fori_loop(0, T1_W // 16, fin, 0)
        pltpu.sync_copy(t1, out_hbm.at[pl.ds(wid * T1_W, T1_W)])

    return k(f0a, f1a, f2a, f3a, f4a, cxa, cya, seg)


def kernel(feat, seg, coords):
    out = _sc_extractor(feat[:, 0], feat[:, 1], feat[:, 2], feat[:, 3],
                        feat[:, 4], coords[:, 0], coords[:, 1],
                        seg.astype(jnp.int32))
    return out.reshape(NSEG, ROW)


# speedup vs baseline: 49.8273x; 1.0057x over previous
import functools

import jax
import jax.numpy as jnp
from jax import lax
from jax.experimental import pallas as pl
from jax.experimental.pallas import tpu as pltpu
from jax.experimental.pallas import tpu_sc as plsc

NB = 8
HALF2D = NB * NB
C_IN = 3
CH = C_IN + 1
INV_S2 = 1.0 / (2.0 * 0.025 * 0.025)
NPTS = 1048576
NSEG = 4096
NW = 32
SEG_W = NSEG // NW
ROW = HALF2D * CH
CHUNK = 4096
PAD = 64
BUF = CHUNK + PAD
NSTEP = CHUNK // 16
T1_W = SEG_W * ROW


def _sc_extractor(f0a, f1a, f2a, f3a, f4a, cxa, cya, seg):
    mesh = plsc.VectorSubcoreMesh(core_axis_name="c", subcore_axis_name="s")

    @functools.partial(
        pl.kernel,
        mesh=mesh,
        out_type=jax.ShapeDtypeStruct((NSEG * ROW,), jnp.float32),
        compiler_params=pltpu.CompilerParams(needs_layout_passes=False,
                                             use_tc_tiling_on_sc=False),
        scratch_types=[
            pltpu.VMEM((BUF,), jnp.int32),
            pltpu.VMEM((BUF,), jnp.float32),
            pltpu.VMEM((BUF,), jnp.float32),
            pltpu.VMEM((BUF,), jnp.float32),
            pltpu.VMEM((BUF,), jnp.float32),
            pltpu.VMEM((BUF,), jnp.float32),
            pltpu.VMEM((BUF,), jnp.float32),
            pltpu.VMEM((BUF,), jnp.float32),
            pltpu.VMEM((T1_W,), jnp.float32),
            pltpu.VMEM((SEG_W * HALF2D,), jnp.float32),
            pltpu.VMEM((16 * SEG_W,), jnp.float32),
            pltpu.VMEM((16 * SEG_W,), jnp.float32),
            pltpu.VMEM((16 * SEG_W,), jnp.float32),
            pltpu.VMEM((16 * SEG_W,), jnp.float32),
            pltpu.VMEM((16 * SEG_W,), jnp.float32),
            pltpu.VMEM((SEG_W,), jnp.float32),
            pltpu.VMEM((SEG_W,), jnp.float32),
            pltpu.VMEM((SEG_W,), jnp.float32),
            pltpu.VMEM((SEG_W,), jnp.float32),
            pltpu.VMEM((SEG_W,), jnp.float32),
            pltpu.VMEM((1024,), jnp.int32),
            pltpu.VMEM((1024,), jnp.int32),
            pltpu.VMEM((1024,), jnp.int32),
            pltpu.SemaphoreType.DMA,
        ],
    )
    def k(f0h, f1h, f2h, f3h, f4h, cxh, cyh, seg_hbm, out_hbm,
          segb, cxb, cyb, f0b, f1b, f2b, f3b, f4b,
          t1, wacc, lox, loy, hix, hiy, cnt16,
          loxr, loyr, dxr, dyr, cntr, sidx, samp, wind, dmasem):
        wid = lax.axis_index("s") * 2 + lax.axis_index("c")
        seg_base = wid * SEG_W
        lane = lax.iota(jnp.int32, 16)
        zf = jnp.zeros((16,), jnp.float32)
        onesf = jnp.ones((16,), jnp.float32)
        zi = jnp.zeros((16,), jnp.int32)
        onesi = jnp.ones((16,), jnp.int32)
        big = jnp.full((16,), 3e38, jnp.float32)

        def bidx(k, _):
            sidx[pl.ds(pl.multiple_of(k * 16, 16), 16)] = (k * 16 + lane) * 1024
            return 0
        lax.fori_loop(0, 64, bidx, 0)
        pltpu.async_copy(seg_hbm.at[sidx], samp, dmasem).wait()

        def count_lt(buf, t):
            def cbody(k, acc):
                v = buf[pl.ds(pl.multiple_of(k * 16, 16), 16)]
                return acc + jnp.where(v < t, 1, 0).astype(jnp.int32)
            acc = lax.fori_loop(0, 64, cbody, jnp.zeros((16,), jnp.int32),
                                unroll=4)
            return jnp.sum(acc)

        def lower_bound(target):
            c = count_lt(samp, target)
            w = jnp.maximum(c - 1, 0) * 1024
            pltpu.sync_copy(seg_hbm.at[pl.ds(pl.multiple_of(w, 8), 1024)],
                            wind)
            return w + count_lt(wind, target)

        p0 = lower_bound(seg_base)
        p1 = lower_bound(seg_base + SEG_W)
        n = p1 - p0
        nchunks = lax.div(n + (CHUNK - 1), CHUNK)

        def z_t1(i, _):
            t1[pl.ds(pl.multiple_of(i * 16, 16), 16)] = zf
            return 0
        lax.fori_loop(0, T1_W // 16, z_t1, 0, unroll=8)

        def z_wacc(i, _):
            wacc[pl.ds(pl.multiple_of(i * 16, 16), 16)] = zf
            return 0
        lax.fori_loop(0, SEG_W * HALF2D // 16, z_wacc, 0, unroll=8)

        def z_lp(i, _):
            sl = pl.ds(pl.multiple_of(i * 16, 16), 16)
            lox[sl] = big
            loy[sl] = big
            hix[sl] = -big
            hiy[sl] = -big
            cnt16[sl] = zf
            return 0
        lax.fori_loop(0, 16 * SEG_W // 16, z_lp, 0, unroll=4)

        def chunk1(kk, _):
            p = p0 + kk * CHUNK
            b8 = jnp.minimum(lax.div(p, 8) * 8, NPTS - BUF)
            b8 = pl.multiple_of(b8, 8)
            shift = p - b8
            pltpu.sync_copy(seg_hbm.at[pl.ds(b8, BUF)], segb)
            pltpu.sync_copy(cxh.at[pl.ds(b8, BUF)], cxb)
            pltpu.sync_copy(cyh.at[pl.ds(b8, BUF)], cyb)
            rem = jnp.minimum(p1 - p, CHUNK)

            def step(j, _):
                rel = lane * NSTEP + j
                valid = rel < rem
                ib = jnp.minimum(rel + shift, BUF - 1)
                sv = plsc.load_gather(segb, [ib])
                sl = jnp.clip(sv - seg_base, 0, SEG_W - 1)
                tidx = lane * SEG_W + sl
                cx = plsc.load_gather(cxb, [ib])
                cy = plsc.load_gather(cyb, [ib])
                cur = plsc.load_gather(lox, [tidx])
                plsc.store_scatter(lox, [tidx], jnp.minimum(cur, cx),
                                   mask=valid)
                cur = plsc.load_gather(loy, [tidx])
                plsc.store_scatter(loy, [tidx], jnp.minimum(cur, cy),
                                   mask=valid)
                cur = plsc.load_gather(hix, [tidx])
                plsc.store_scatter(hix, [tidx], jnp.maximum(cur, cx),
                                   mask=valid)
                cur = plsc.load_gather(hiy, [tidx])
                plsc.store_scatter(hiy, [tidx], jnp.maximum(cur, cy),
                                   mask=valid)
                plsc.addupdate_scatter(cnt16, [tidx], onesf, mask=valid)
                return 0

            lax.fori_loop(0, NSTEP, step, 0, unroll=2)
            return 0

        lax.fori_loop(0, nchunks, chunk1, 0)

        for blk in range(SEG_W // 16):
            o = blk * 16
            mnx = lox[pl.ds(o, 16)]
            mny = loy[pl.ds(o, 16)]
            mxx = hix[pl.ds(o, 16)]
            mxy = hiy[pl.ds(o, 16)]
            csum = cnt16[pl.ds(o, 16)]
            for l in range(1, 16):
                q = l * SEG_W + o
                mnx = jnp.minimum(mnx, lox[pl.ds(q, 16)])
                mny = jnp.minimum(mny, loy[pl.ds(q, 16)])
                mxx = jnp.maximum(mxx, hix[pl.ds(q, 16)])
                mxy = jnp.maximum(mxy, hiy[pl.ds(q, 16)])
                csum = csum + cnt16[pl.ds(q, 16)]
            loxr[pl.ds(o, 16)] = mnx
            loyr[pl.ds(o, 16)] = mny
            dxr[pl.ds(o, 16)] = mxx - mnx + 1e-6
            dyr[pl.ds(o, 16)] = mxy - mny + 1e-6
            cntr[pl.ds(o, 16)] = csum

        def chunk2(kk, _):
            p = p0 + kk * CHUNK
            b8 = jnp.minimum(lax.div(p, 8) * 8, NPTS - BUF)
            b8 = pl.multiple_of(b8, 8)
            shift = p - b8
            pltpu.sync_copy(seg_hbm.at[pl.ds(b8, BUF)], segb)
            pltpu.sync_copy(cxh.at[pl.ds(b8, BUF)], cxb)
            pltpu.sync_copy(cyh.at[pl.ds(b8, BUF)], cyb)
            pltpu.sync_copy(f0h.at[pl.ds(b8, BUF)], f0b)
            pltpu.sync_copy(f1h.at[pl.ds(b8, BUF)], f1b)
            pltpu.sync_copy(f2h.at[pl.ds(b8, BUF)], f2b)
            pltpu.sync_copy(f3h.at[pl.ds(b8, BUF)], f3b)
            pltpu.sync_copy(f4h.at[pl.ds(b8, BUF)], f4b)
            rem = jnp.minimum(p1 - p, CHUNK)

            def step(j, _):
                rel = j * 16 + lane
                valid = rel < rem
                ib = jnp.minimum(rel + shift, BUF - 1)
                sv = plsc.load_gather(segb, [ib])
                sl = jnp.clip(sv - seg_base, 0, SEG_W - 1)
                cx = plsc.load_gather(cxb, [ib])
                cy = plsc.load_gather(cyb, [ib])
                lx = plsc.load_gather(loxr, [sl])
                ly = plsc.load_gather(loyr, [sl])
                dx = plsc.load_gather(dxr, [sl])
                dy = plsc.load_gather(dyr, [sl])
                gx = (cx - lx) / dx * 7.0
                gy = (cy - ly) / dy * 7.0
                i0x = jnp.minimum(gx.astype(jnp.int32), 6)
                i0y = jnp.minimum(gy.astype(jnp.int32), 6)
                fx = gx - i0x.astype(jnp.float32)
                fy = gy - i0y.astype(jnp.float32)
                wx0 = 1.0 - fx
                wy0 = 1.0 - fy
                w00 = wx0 * wy0
                w01 = wx0 * fy
                w10 = fx * wy0
                w11 = fx * fy
                base = sl * ROW + i0x * (NB * CH) + i0y * CH
                f0 = plsc.load_gather(f0b, [ib])
                f1 = plsc.load_gather(f1b, [ib])
                f2 = plsc.load_gather(f2b, [ib])
                for off, w in ((0, w00), (CH, w01), (NB * CH, w10),
                               (NB * CH + CH, w11)):
                    bb = base + off
                    plsc.addupdate_scatter(t1, [bb], f0 * w, mask=valid)
                    plsc.addupdate_scatter(t1, [bb + 1], f1 * w, mask=valid)
                    plsc.addupdate_scatter(t1, [bb + 2], f2 * w, mask=valid)
                    plsc.addupdate_scatter(wacc, [lax.shift_right_logical(bb, 2)],
                                           w, mask=valid)
                f3 = plsc.load_gather(f3b, [ib])
                f4 = plsc.load_gather(f4b, [ib])
                g3 = f3 * 8.0 - 0.5
                g4 = f4 * 8.0 - 0.5
                i3f = (g3 + 8.0).astype(jnp.int32).astype(jnp.float32) - 8.0
                i4f = (g4 + 8.0).astype(jnp.int32).astype(jnp.float32) - 8.0
                fr3 = g3 - i3f
                fr4 = g4 - i4f
                wa3 = 1.0 / (1.0 + jnp.exp((2.0 * fr3 - 1.0) * INV_S2))
                wa4 = 1.0 / (1.0 + jnp.exp((2.0 * fr4 - 1.0) * INV_S2))
                wb3 = 1.0 - wa3
                wb4 = 1.0 - wa4
                i3 = i3f.astype(jnp.int32)
                i4 = i4f.astype(jnp.int32)
                a3 = jnp.clip(i3, 0, NB - 1) * (NB * CH)
                b3 = jnp.clip(i3 + 1, 0, NB - 1) * (NB * CH)
                a4 = jnp.clip(i4, 0, NB - 1) * CH
                b4 = jnp.clip(i4 + 1, 0, NB - 1) * CH
                hb = sl * ROW + 3
                plsc.addupdate_scatter(t1, [hb + a3 + a4], wa3 * wa4,
                                       mask=valid)
                plsc.addupdate_scatter(t1, [hb + a3 + b4], wa3 * wb4,
                                       mask=valid)
                plsc.addupdate_scatter(t1, [hb + b3 + a4], wb3 * wa4,
                                       mask=valid)
                plsc.addupdate_scatter(t1, [hb + b3 + b4], wb3 * wb4,
                                       mask=valid)
                return 0

            lax.fori_loop(0, NSTEP, step, 0, unroll=2)
            return 0

        lax.fori_loop(0, nchunks, chunk2, 0)

        ch_is_hist = (lane & 3) == 3

        def fin(kk, _):
            idxv = kk * 16 + lane
            wv = plsc.load_gather(wacc, [lax.shift_right_logical(idxv, 2)])
            cv = plsc.load_gather(cntr, [lax.shift_right_logical(idxv, 8)])
            den = jnp.where(ch_is_hist, cv, wv) + 1e-6
            v = plsc.load_gather(t1, [idxv])
            plsc.store_scatter(t1, [idxv], v / den)
            return 0

        lax.fori_loop(0, T1_W // 16, fin, 0, unroll=4)
        pltpu.sync_copy(t1, out_hbm.at[pl.ds(wid * T1_W, T1_W)])

    return k(f0a, f1a, f2a, f3a, f4a, cxa, cya, seg)


def kernel(feat, seg, coords):
    out = _sc_extractor(feat[:, 0], feat[:, 1], feat[:, 2], feat[:, 3],
                        feat[:, 4], coords[:, 0], coords[:, 1],
                        seg.astype(jnp.int32))
    return out.reshape(NSEG, ROW)
